# Initial kernel scaffold; baseline (speedup 1.0000x reference)
import jax
import jax.numpy as jnp
from jax.experimental import pallas as pl


def kernel(x, edge_index, batch, lin_in_W, lin_in_b, msg_W1, msg_b1, msg_W2, msg_b2, upd_W1, upd_b1, upd_W2, upd_b2, pred_W, pred_b):
    raise NotImplementedError("write your pallas kernel here")


# speedup vs baseline: 1.9501x; 1.9501x over previous
import functools

import jax
import jax.numpy as jnp
from jax import lax
from jax.experimental import pallas as pl
from jax.experimental.pallas import tpu as pltpu
from jax.experimental.pallas import tpu_sc as plsc

_N = 50000
_E = 800000
_EMB = 64
_L = 4

_NPAD = 50176
_EPAD = 802816
_ROWB = 1024
_EROWB = 2048

_mesh = plsc.VectorSubcoreMesh(core_axis_name="c", subcore_axis_name="s")

_GBLK = 128
_GNBLK = _EPAD // (32 * _GBLK)


@functools.partial(
    pl.kernel,
    out_type=jax.ShapeDtypeStruct((_EPAD, 128), jnp.float32),
    mesh=_mesh,
    scratch_types=[
        pltpu.VMEM((1, 128), jnp.int32),
        pltpu.VMEM((1, 128), jnp.int32),
        pltpu.VMEM((1, 128), jnp.int32),
        pltpu.VMEM((1, 128), jnp.int32),
        pltpu.VMEM((_GBLK, 128), jnp.float32),
        pltpu.VMEM((_GBLK, 128), jnp.float32),
        pltpu.VMEM((_GBLK, 128), jnp.float32),
        pltpu.VMEM((_GBLK, 128), jnp.float32),
        pltpu.VMEM((_GBLK, 128), jnp.float32),
        pltpu.VMEM((_GBLK, 128), jnp.float32),
        pltpu.SemaphoreType.DMA,
        pltpu.SemaphoreType.DMA,
        pltpu.SemaphoreType.DMA,
        pltpu.SemaphoreType.DMA,
        pltpu.SemaphoreType.DMA,
        pltpu.SemaphoreType.DMA,
    ],
)
def _gather_msg(t_hbm, dst_hbm, src_hbm, out_hbm,
                idxd0, idxs0, idxd1, idxs1,
                bufd0, bufs0, bufd1, bufs1, bufo0, bufo1,
                sa0, sb0, sa1, sb1, sw0, sw1):
    c = lax.axis_index("c")
    s = lax.axis_index("s")
    wid = s * 2 + c
    base = wid * _GNBLK
    sets = ((idxd0, idxs0, bufd0, bufs0, bufo0, sa0, sb0, sw0),
            (idxd1, idxs1, bufd1, bufs1, bufo1, sa1, sb1, sw1))

    def prep(step, st):
        idxd, idxs, bufd, bufs, _, sa, sb, _2 = st
        ch = base + step
        pltpu.sync_copy(dst_hbm.at[pl.ds(ch, 1)], idxd)
        pltpu.sync_copy(src_hbm.at[pl.ds(ch, 1)], idxs)
        pltpu.async_copy(t_hbm.at[idxd.at[0]], bufd, sa)
        pltpu.async_copy(t_hbm.at[idxs.at[0]], bufs, sb)

    prep(0, sets[0])
    prep(1, sets[1])

    def outer(i2, carry):
        for p in range(2):
            st = sets[p]
            idxd, idxs, bufd, bufs, bufo, sa, sb, sw = st
            i = 2 * i2 + p
            ch = base + i

            @pl.when(i >= 2)
            def _():
                pltpu.make_async_copy(bufo, out_hbm.at[pl.ds(0, _GBLK)],
                                      sw).wait()

            pltpu.make_async_copy(t_hbm.at[idxd.at[0]], bufd, sa).wait()
            pltpu.make_async_copy(t_hbm.at[idxs.at[0]], bufs, sb).wait()

            def row(r, _):
                for g in range(0, _EMB, 16):
                    va = bufd[r, pl.ds(g, 16)]
                    vb = bufs[r, pl.ds(_EMB + g, 16)]
                    bufo[r, pl.ds(g, 16)] = jnp.maximum(va + vb, 0.0)
                return 0

            lax.fori_loop(0, _GBLK, row, 0)
            pltpu.async_copy(bufo, out_hbm.at[pl.ds(ch * _GBLK, _GBLK)], sw)

            @pl.when(i + 2 < _GNBLK)
            def _():
                prep(i + 2, st)
        return carry

    lax.fori_loop(0, _GNBLK // 2, outer, 0)
    pltpu.make_async_copy(bufo0, out_hbm.at[pl.ds(0, _GBLK)], sw0).wait()
    pltpu.make_async_copy(bufo1, out_hbm.at[pl.ds(0, _GBLK)], sw1).wait()


_SBLK = 128
_SNBLK = _EPAD // (16 * _SBLK)
_AROWS = _NPAD // 4
_TAROWS = _AROWS // 16


@functools.partial(
    pl.kernel,
    out_type=jax.ShapeDtypeStruct((2, _AROWS, 128), jnp.float32),
    mesh=_mesh,
    scratch_types=[
        pltpu.VMEM((1, 128), jnp.int32),
        pltpu.VMEM((1, 128), jnp.int32),
        pltpu.VMEM((_SBLK, 128), jnp.float32),
        pltpu.VMEM_SHARED((_AROWS, 128), jnp.float32),
        pltpu.SemaphoreType.DMA,
    ],
)
def _scatter_aggr(p0_hbm, p1_hbm, dst_hbm, out_hbm, idxd, rbuf, bufp,
                  acc_sh, sem):
    c = lax.axis_index("c")
    s = lax.axis_index("s")

    def zrow(r, _):
        for g in range(0, 128, 16):
            bufp[r, pl.ds(g, 16)] = jnp.zeros((16,), jnp.float32)
        return 0

    lax.fori_loop(0, _SBLK, zrow, 0)
    row0 = s * _TAROWS

    def zcp(k, _):
        pltpu.sync_copy(bufp.at[pl.ds(0, 112)],
                        acc_sh.at[pl.ds(pl.multiple_of(row0 + k * 112, 8),
                                        112)])
        return 0

    lax.fori_loop(0, 7, zcp, 0)
    plsc.subcore_barrier()

    def one_core(p_hbm):
        def block(bi, carry):
            ch0 = s * _SNBLK + bi
            e0 = ch0 * 128
            pltpu.sync_copy(dst_hbm.at[pl.ds(ch0, 1)], idxd)
            pltpu.sync_copy(p_hbm.at[pl.ds(e0, _SBLK)], bufp)
            for g in range(0, 128, 16):
                v = idxd[0, pl.ds(g, 16)]
                rbuf[0, pl.ds(g, 16)] = lax.shift_right_logical(v, 2)
            pltpu.sync_copy(bufp, acc_sh.at[rbuf.at[0]], add=True)
            return carry

        lax.fori_loop(0, _SNBLK, block, 0)

    @pl.when(c == 0)
    def _():
        one_core(p0_hbm)

    @pl.when(c == 1)
    def _():
        one_core(p1_hbm)

    plsc.subcore_barrier()

    def ocp(k, _):
        r = pl.multiple_of(row0 + k * 112, 8)
        pltpu.sync_copy(acc_sh.at[pl.ds(r, 112)],
                        out_hbm.at[c].at[pl.ds(r, 112)])
        return 0

    lax.fori_loop(0, 7, ocp, 0)


def _lin_in_body(x_ref, w_ref, b_ref, o_ref):
    o_ref[...] = (jnp.dot(x_ref[...], w_ref[...],
                          preferred_element_type=jnp.float32) + b_ref[...])


def _stage_a_body(h_ref, wa_ref, wb_ref, b_ref, t_ref):
    h = h_ref[...]
    a = (jnp.dot(h, wa_ref[...], preferred_element_type=jnp.float32)
         + b_ref[...])
    b = jnp.dot(h, wb_ref[...], preferred_element_type=jnp.float32)
    t_ref[...] = jnp.concatenate([a, b], axis=1)


def _stage_c_body(m1_ref, s1_ref, s2_ref, w_ref, b_ref, o0_ref, o1_ref):
    m1 = m1_ref[:, :_EMB]
    m = jnp.maximum(
        jnp.dot(m1, w_ref[...], preferred_element_type=jnp.float32)
        + b_ref[...], 0.0)
    s1 = s1_ref[...]
    s2 = s2_ref[...]
    s3 = s1 * s2
    for c, o_ref in ((0, o0_ref), (1, o1_ref)):
        half = m[:, 32 * c:32 * c + 32]
        o_ref[...] = jnp.concatenate(
            [half, half * s1, half * s2, half * s3], axis=1)


def _unmix_body(acc_ref, hm_ref, o_ref):
    o_ref[...] = jnp.dot(acc_ref[...], hm_ref[...],
                         preferred_element_type=jnp.float32)


def _stage_e_body(h_ref, a0_ref, a1_ref, ua_ref, ub_ref, c1_ref, u2_ref,
                  c2_ref, o_ref):
    h = h_ref[...]
    ag = jnp.concatenate([a0_ref[...], a1_ref[...]], axis=1)
    u1 = jnp.maximum(
        jnp.dot(h, ua_ref[...], preferred_element_type=jnp.float32)
        + jnp.dot(ag, ub_ref[...], preferred_element_type=jnp.float32)
        + c1_ref[...], 0.0)
    u = jnp.maximum(
        jnp.dot(u1, u2_ref[...], preferred_element_type=jnp.float32)
        + c2_ref[...], 0.0)
    o_ref[...] = h + u


def _pool_body(h_ref, pw_ref, pb_ref, o_ref, acc_ref):
    i = pl.program_id(0)

    @pl.when(i == 0)
    def _():
        acc_ref[...] = jnp.zeros_like(acc_ref)

    rows = jax.lax.broadcasted_iota(jnp.int32, (_ROWB, 1), 0) + i * _ROWB
    mask = (rows < _N).astype(jnp.float32)
    acc_ref[...] += jnp.sum(h_ref[...] * mask, axis=0, keepdims=True)

    @pl.when(i == pl.num_programs(0) - 1)
    def _():
        o_ref[...] = (jnp.dot(acc_ref[...] / float(_N), pw_ref[...],
                              preferred_element_type=jnp.float32)
                      + pb_ref[...])


def _row_spec(blk, ncol):
    return pl.BlockSpec((blk, ncol), lambda i: (i, 0))


def _full_spec(shape):
    return pl.BlockSpec(shape, lambda i: tuple(0 for _ in shape))


def _lin_in(xp, w, b):
    return pl.pallas_call(
        _lin_in_body,
        grid=(_NPAD // _ROWB,),
        in_specs=[_row_spec(_ROWB, 8), _full_spec((8, _EMB)),
                  _full_spec((1, _EMB))],
        out_specs=_row_spec(_ROWB, _EMB),
        out_shape=jax.ShapeDtypeStruct((_NPAD, _EMB), jnp.float32),
    )(xp, w, b)


def _stage_a(h, wa, wb, b):
    return pl.pallas_call(
        _stage_a_body,
        grid=(_NPAD // _ROWB,),
        in_specs=[_row_spec(_ROWB, _EMB), _full_spec((_EMB, _EMB)),
                  _full_spec((_EMB, _EMB)), _full_spec((1, _EMB))],
        out_specs=_row_spec(_ROWB, 2 * _EMB),
        out_shape=jax.ShapeDtypeStruct((_NPAD, 2 * _EMB), jnp.float32),
    )(h, wa, wb, b)


def _stage_c(m1x, sg1, sg2, w, b):
    return pl.pallas_call(
        _stage_c_body,
        grid=(_EPAD // _EROWB,),
        in_specs=[_row_spec(_EROWB, 128), _row_spec(_EROWB, 1),
                  _row_spec(_EROWB, 1),
                  _full_spec((_EMB, _EMB)), _full_spec((1, _EMB))],
        out_specs=[_row_spec(_EROWB, 128), _row_spec(_EROWB, 128)],
        out_shape=[jax.ShapeDtypeStruct((_EPAD, 128), jnp.float32),
                   jax.ShapeDtypeStruct((_EPAD, 128), jnp.float32)],
    )(m1x, sg1, sg2, w, b)


def _unmix(acc2, hm):
    return pl.pallas_call(
        _unmix_body,
        grid=(2 * _AROWS // 896,),
        in_specs=[_row_spec(896, 128), _full_spec((128, 128))],
        out_specs=_row_spec(896, 128),
        out_shape=jax.ShapeDtypeStruct((2 * _AROWS, 128), jnp.float32),
    )(acc2, hm)


def _stage_e(h, a0, a1, ua, ub, c1, u2, c2):
    return pl.pallas_call(
        _stage_e_body,
        grid=(_NPAD // _ROWB,),
        in_specs=[_row_spec(_ROWB, _EMB), _row_spec(_ROWB, 32),
                  _row_spec(_ROWB, 32),
                  _full_spec((_EMB, _EMB)), _full_spec((_EMB, _EMB)),
                  _full_spec((1, _EMB)), _full_spec((_EMB, _EMB)),
                  _full_spec((1, _EMB))],
        out_specs=_row_spec(_ROWB, _EMB),
        out_shape=jax.ShapeDtypeStruct((_NPAD, _EMB), jnp.float32),
    )(h, a0, a1, ua, ub, c1, u2, c2)


def _pool(h, pw, pb):
    return pl.pallas_call(
        _pool_body,
        grid=(_NPAD // _ROWB,),
        in_specs=[_row_spec(_ROWB, _EMB), _full_spec((_EMB, 1)),
                  _full_spec((1, 1))],
        out_specs=_full_spec((1, 1)),
        out_shape=jax.ShapeDtypeStruct((1, 1), jnp.float32),
        scratch_shapes=[pltpu.VMEM((1, _EMB), jnp.float32)],
    )(h, pw, pb)


def kernel(x, edge_index, batch, lin_in_W, lin_in_b,
           msg_W1, msg_b1, msg_W2, msg_b2,
           upd_W1, upd_b1, upd_W2, upd_b2,
           pred_W, pred_b):
    src = edge_index[0]
    dst = edge_index[1]
    pad_e = _EPAD - _E
    dstp = jnp.concatenate(
        [dst, jnp.full((pad_e,), _N, jnp.int32)]).reshape(_EPAD // 128, 128)
    srcp = jnp.concatenate(
        [src, jnp.full((pad_e,), _N, jnp.int32)]).reshape(_EPAD // 128, 128)
    sg1 = (1 - 2 * (dstp & 1)).astype(jnp.float32).reshape(_EPAD, 1)
    sg2 = (1 - (dstp & 2)).astype(jnp.float32).reshape(_EPAD, 1)
    import numpy as _np
    h4 = _np.array([[1, 1, 1, 1], [1, -1, 1, -1],
                    [1, 1, -1, -1], [1, -1, -1, 1]], dtype=_np.float32)
    hm = jnp.asarray(_np.kron(h4 / 4.0, _np.eye(32, dtype=_np.float32)))

    xp = jnp.pad(x, ((0, _NPAD - _N), (0, 1)))
    w_in = jnp.pad(lin_in_W, ((0, 1), (0, 0)))

    emb = _lin_in(xp, w_in, lin_in_b.reshape(1, -1))
    for l in range(_L):
        t_tab = _stage_a(emb, msg_W1[l, :_EMB], msg_W1[l, _EMB:],
                         msg_b1[l].reshape(1, -1))
        m1x = _gather_msg(t_tab, dstp, srcp)
        p0, p1 = _stage_c(m1x, sg1, sg2, msg_W2[l],
                          msg_b2[l].reshape(1, -1))
        aggr = _scatter_aggr(p0, p1, dstp)
        unm = _unmix(aggr.reshape(2 * _AROWS, 128), hm)
        a0 = unm[:_AROWS].reshape(_NPAD, 32)
        a1 = unm[_AROWS:].reshape(_NPAD, 32)
        emb = _stage_e(emb, a0, a1, upd_W1[l, :_EMB], upd_W1[l, _EMB:],
                       upd_b1[l].reshape(1, -1), upd_W2[l],
                       upd_b2[l].reshape(1, -1))
    out = _pool(emb, pred_W, pred_b.reshape(1, 1))
    return out.reshape(-1)


# speedup vs baseline: 1.9825x; 1.0166x over previous
import functools

import jax
import jax.numpy as jnp
from jax import lax
from jax.experimental import pallas as pl
from jax.experimental.pallas import tpu as pltpu
from jax.experimental.pallas import tpu_sc as plsc

_N = 50000
_E = 800000
_EMB = 64
_L = 4

_NPAD = 50176
_EPAD = 802816
_ROWB = 1024
_EROWB = 2048

_mesh = plsc.VectorSubcoreMesh(core_axis_name="c", subcore_axis_name="s")

_GBLK = 128
_GNBLK = _EPAD // (32 * _GBLK)


@functools.partial(
    pl.kernel,
    out_type=jax.ShapeDtypeStruct((_EPAD, 128), jnp.float32),
    mesh=_mesh,
    scratch_types=[
        pltpu.VMEM((1, 2, 128), jnp.int32),
        pltpu.VMEM((1, 2, 128), jnp.int32),
        pltpu.VMEM((_GBLK, 128), jnp.float32),
        pltpu.VMEM((_GBLK, 128), jnp.float32),
        pltpu.VMEM((_GBLK, 128), jnp.float32),
        pltpu.VMEM((_GBLK, 128), jnp.float32),
        pltpu.VMEM((_GBLK, 128), jnp.float32),
        pltpu.VMEM((_GBLK, 128), jnp.float32),
        pltpu.SemaphoreType.DMA,
        pltpu.SemaphoreType.DMA,
        pltpu.SemaphoreType.DMA,
        pltpu.SemaphoreType.DMA,
        pltpu.SemaphoreType.DMA,
        pltpu.SemaphoreType.DMA,
    ],
)
def _gather_msg(t_hbm, dsil_hbm, out_hbm,
                idxd0, idxd1,
                bufd0, bufs0, bufd1, bufs1, bufo0, bufo1,
                sa0, sb0, sa1, sb1, sw0, sw1):
    c = lax.axis_index("c")
    s = lax.axis_index("s")
    wid = s * 2 + c
    base = wid * _GNBLK
    sets = ((idxd0, bufd0, bufs0, bufo0, sa0, sb0, sw0),
            (idxd1, bufd1, bufs1, bufo1, sa1, sb1, sw1))

    def prep(step, st):
        idxd, bufd, bufs, _, sa, sb, _2 = st
        ch = base + step
        pltpu.sync_copy(dsil_hbm.at[pl.ds(ch, 1)], idxd)
        pltpu.async_copy(t_hbm.at[idxd.at[0, 0]], bufd, sa)
        pltpu.async_copy(t_hbm.at[idxd.at[0, 1]], bufs, sb)

    prep(0, sets[0])
    prep(1, sets[1])

    def outer(i2, carry):
        for p in range(2):
            st = sets[p]
            idxd, bufd, bufs, bufo, sa, sb, sw = st
            i = 2 * i2 + p
            ch = base + i

            @pl.when(i >= 2)
            def _():
                pltpu.make_async_copy(bufo, out_hbm.at[pl.ds(0, _GBLK)],
                                      sw).wait()

            pltpu.make_async_copy(t_hbm.at[idxd.at[0, 0]], bufd, sa).wait()
            pltpu.make_async_copy(t_hbm.at[idxd.at[0, 1]], bufs, sb).wait()

            @plsc.parallel_loop(0, _GBLK, unroll=8)
            def _(r):
                for g in range(0, _EMB, 16):
                    va = bufd[r, pl.ds(g, 16)]
                    vb = bufs[r, pl.ds(_EMB + g, 16)]
                    bufo[r, pl.ds(g, 16)] = jnp.maximum(va + vb, 0.0)
            pltpu.async_copy(bufo, out_hbm.at[pl.ds(ch * _GBLK, _GBLK)], sw)

            @pl.when(i + 2 < _GNBLK)
            def _():
                prep(i + 2, st)
        return carry

    lax.fori_loop(0, _GNBLK // 2, outer, 0)
    pltpu.make_async_copy(bufo0, out_hbm.at[pl.ds(0, _GBLK)], sw0).wait()
    pltpu.make_async_copy(bufo1, out_hbm.at[pl.ds(0, _GBLK)], sw1).wait()


_SBLK = 128
_SNBLK = _EPAD // (16 * _SBLK)
_AROWS = _NPAD // 4
_TAROWS = _AROWS // 16


@functools.partial(
    pl.kernel,
    out_type=jax.ShapeDtypeStruct((2, _AROWS, 128), jnp.float32),
    mesh=_mesh,
    scratch_types=[
        pltpu.VMEM((1, 4, 128), jnp.int32),
        pltpu.VMEM((_SBLK, 128), jnp.float32),
        pltpu.VMEM_SHARED((_AROWS, 128), jnp.float32),
        pltpu.SemaphoreType.DMA,
    ],
)
def _scatter_aggr(p0_hbm, p1_hbm, rows_hbm, out_hbm, idxd, bufp0,
                  acc_sh, sl0):
    c = lax.axis_index("c")
    s = lax.axis_index("s")

    def zrow(r, _):
        for g in range(0, 128, 16):
            bufp0[r, pl.ds(g, 16)] = jnp.zeros((16,), jnp.float32)
        return 0

    lax.fori_loop(0, _SBLK, zrow, 0)
    row0 = s * _TAROWS

    def zcp(k, _):
        pltpu.sync_copy(bufp0.at[pl.ds(0, 16)],
                        acc_sh.at[pl.ds(pl.multiple_of(row0 + k * 16, 8),
                                        16)])
        return 0

    lax.fori_loop(0, 49, zcp, 0)
    plsc.subcore_barrier()

    def one_core(p_hbm):
        base = s * _SNBLK

        def block(bi, carry):
            ch = base + bi
            e0 = ch * 128

            @pl.when(bi % 4 == 0)
            def _():
                pltpu.sync_copy(rows_hbm.at[pl.ds(ch // 4, 1)], idxd)

            pltpu.sync_copy(p_hbm.at[pl.ds(e0, _SBLK)], bufp0)
            pltpu.sync_copy(bufp0, acc_sh.at[idxd.at[0, bi % 4]], add=True)
            return carry

        lax.fori_loop(0, _SNBLK, block, 0)

    @pl.when(c == 0)
    def _():
        one_core(p0_hbm)

    @pl.when(c == 1)
    def _():
        one_core(p1_hbm)

    plsc.subcore_barrier()

    def ocp(k, _):
        r = pl.multiple_of(row0 + k * 16, 8)
        pltpu.sync_copy(acc_sh.at[pl.ds(r, 16)],
                        out_hbm.at[c].at[pl.ds(r, 16)])
        return 0

    lax.fori_loop(0, 49, ocp, 0)


def _lin_in_body(x_ref, w_ref, b_ref, o_ref):
    o_ref[...] = (jnp.dot(x_ref[...], w_ref[...],
                          preferred_element_type=jnp.float32,
                precision=lax.Precision.HIGHEST) + b_ref[...])


def _stage_a_body(h_ref, wa_ref, wb_ref, b_ref, t_ref):
    h = h_ref[...]
    a = (jnp.dot(h, wa_ref[...], preferred_element_type=jnp.float32,
                precision=lax.Precision.HIGHEST)
         + b_ref[...])
    b = jnp.dot(h, wb_ref[...], preferred_element_type=jnp.float32,
                precision=lax.Precision.HIGHEST)
    t_ref[...] = jnp.concatenate([a, b], axis=1)


def _stage_c_body(m1_ref, s1_ref, s2_ref, w_ref, b_ref, o0_ref, o1_ref):
    m1 = m1_ref[:, :_EMB]
    m = jnp.maximum(
        jnp.dot(m1, w_ref[...], preferred_element_type=jnp.float32,
                precision=lax.Precision.HIGHEST)
        + b_ref[...], 0.0)
    s1 = s1_ref[...]
    s2 = s2_ref[...]
    s3 = s1 * s2
    for c, o_ref in ((0, o0_ref), (1, o1_ref)):
        half = m[:, 32 * c:32 * c + 32]
        o_ref[...] = jnp.concatenate(
            [half, half * s1, half * s2, half * s3], axis=1)


def _unmix_body(acc_ref, hm_ref, o_ref):
    o_ref[...] = jnp.dot(acc_ref[...], hm_ref[...],
                         preferred_element_type=jnp.float32,
                precision=lax.Precision.HIGHEST)


def _stage_e_body(h_ref, a0_ref, a1_ref, ua_ref, ub_ref, c1_ref, u2_ref,
                  c2_ref, o_ref):
    h = h_ref[...]
    ag = jnp.concatenate([a0_ref[...], a1_ref[...]], axis=1)
    u1 = jnp.maximum(
        jnp.dot(h, ua_ref[...], preferred_element_type=jnp.float32,
                precision=lax.Precision.HIGHEST)
        + jnp.dot(ag, ub_ref[...], preferred_element_type=jnp.float32,
                precision=lax.Precision.HIGHEST)
        + c1_ref[...], 0.0)
    u = jnp.maximum(
        jnp.dot(u1, u2_ref[...], preferred_element_type=jnp.float32,
                precision=lax.Precision.HIGHEST)
        + c2_ref[...], 0.0)
    o_ref[...] = h + u


def _pool_body(h_ref, pw_ref, pb_ref, o_ref, acc_ref):
    i = pl.program_id(0)

    @pl.when(i == 0)
    def _():
        acc_ref[...] = jnp.zeros_like(acc_ref)

    rows = jax.lax.broadcasted_iota(jnp.int32, (_ROWB, 1), 0) + i * _ROWB
    mask = (rows < _N).astype(jnp.float32)
    acc_ref[...] += jnp.sum(h_ref[...] * mask, axis=0, keepdims=True)

    @pl.when(i == pl.num_programs(0) - 1)
    def _():
        o_ref[...] = (jnp.dot(acc_ref[...] / float(_N), pw_ref[...],
                              preferred_element_type=jnp.float32,
                precision=lax.Precision.HIGHEST)
                      + pb_ref[...])


def _row_spec(blk, ncol):
    return pl.BlockSpec((blk, ncol), lambda i: (i, 0))


def _full_spec(shape):
    return pl.BlockSpec(shape, lambda i: tuple(0 for _ in shape))


def _lin_in(xp, w, b):
    return pl.pallas_call(
        _lin_in_body,
        grid=(_NPAD // _ROWB,),
        in_specs=[_row_spec(_ROWB, 8), _full_spec((8, _EMB)),
                  _full_spec((1, _EMB))],
        out_specs=_row_spec(_ROWB, _EMB),
        out_shape=jax.ShapeDtypeStruct((_NPAD, _EMB), jnp.float32),
    )(xp, w, b)


def _stage_a(h, wa, wb, b):
    return pl.pallas_call(
        _stage_a_body,
        grid=(_NPAD // _ROWB,),
        in_specs=[_row_spec(_ROWB, _EMB), _full_spec((_EMB, _EMB)),
                  _full_spec((_EMB, _EMB)), _full_spec((1, _EMB))],
        out_specs=_row_spec(_ROWB, 2 * _EMB),
        out_shape=jax.ShapeDtypeStruct((_NPAD, 2 * _EMB), jnp.float32),
    )(h, wa, wb, b)


def _stage_c(m1x, sg1, sg2, w, b):
    return pl.pallas_call(
        _stage_c_body,
        grid=(_EPAD // _EROWB,),
        in_specs=[_row_spec(_EROWB, 128), _row_spec(_EROWB, 1),
                  _row_spec(_EROWB, 1),
                  _full_spec((_EMB, _EMB)), _full_spec((1, _EMB))],
        out_specs=[_row_spec(_EROWB, 128), _row_spec(_EROWB, 128)],
        out_shape=[jax.ShapeDtypeStruct((_EPAD, 128), jnp.float32),
                   jax.ShapeDtypeStruct((_EPAD, 128), jnp.float32)],
    )(m1x, sg1, sg2, w, b)


def _unmix(acc2, hm):
    return pl.pallas_call(
        _unmix_body,
        grid=(2 * _AROWS // 896,),
        in_specs=[_row_spec(896, 128), _full_spec((128, 128))],
        out_specs=_row_spec(896, 128),
        out_shape=jax.ShapeDtypeStruct((2 * _AROWS, 128), jnp.float32),
    )(acc2, hm)


def _stage_e(h, a0, a1, ua, ub, c1, u2, c2):
    return pl.pallas_call(
        _stage_e_body,
        grid=(_NPAD // _ROWB,),
        in_specs=[_row_spec(_ROWB, _EMB), _row_spec(_ROWB, 32),
                  _row_spec(_ROWB, 32),
                  _full_spec((_EMB, _EMB)), _full_spec((_EMB, _EMB)),
                  _full_spec((1, _EMB)), _full_spec((_EMB, _EMB)),
                  _full_spec((1, _EMB))],
        out_specs=_row_spec(_ROWB, _EMB),
        out_shape=jax.ShapeDtypeStruct((_NPAD, _EMB), jnp.float32),
    )(h, a0, a1, ua, ub, c1, u2, c2)


def _pool(h, pw, pb):
    return pl.pallas_call(
        _pool_body,
        grid=(_NPAD // _ROWB,),
        in_specs=[_row_spec(_ROWB, _EMB), _full_spec((_EMB, 1)),
                  _full_spec((1, 1))],
        out_specs=_full_spec((1, 1)),
        out_shape=jax.ShapeDtypeStruct((1, 1), jnp.float32),
        scratch_shapes=[pltpu.VMEM((1, _EMB), jnp.float32)],
    )(h, pw, pb)


def kernel(x, edge_index, batch, lin_in_W, lin_in_b,
           msg_W1, msg_b1, msg_W2, msg_b2,
           upd_W1, upd_b1, upd_W2, upd_b2,
           pred_W, pred_b):
    src = edge_index[0]
    dst = edge_index[1]
    pad_e = _EPAD - _E
    dstp = jnp.concatenate(
        [dst, jnp.full((pad_e,), _N, jnp.int32)]).reshape(_EPAD // 128, 128)
    srcp = jnp.concatenate(
        [src, jnp.full((pad_e,), _N, jnp.int32)]).reshape(_EPAD // 128, 128)
    dsil = jnp.stack([dstp, srcp], axis=1)
    rows4 = (dstp >> 2).reshape(_EPAD // 512, 4, 128)
    sg1 = (1 - 2 * (dstp & 1)).astype(jnp.float32).reshape(_EPAD, 1)
    sg2 = (1 - (dstp & 2)).astype(jnp.float32).reshape(_EPAD, 1)
    import numpy as _np
    h4 = _np.array([[1, 1, 1, 1], [1, -1, 1, -1],
                    [1, 1, -1, -1], [1, -1, -1, 1]], dtype=_np.float32)
    hm = jnp.asarray(_np.kron(h4 / 4.0, _np.eye(32, dtype=_np.float32)))

    xp = jnp.pad(x, ((0, _NPAD - _N), (0, 1)))
    w_in = jnp.pad(lin_in_W, ((0, 1), (0, 0)))

    emb = _lin_in(xp, w_in, lin_in_b.reshape(1, -1))
    for l in range(_L):
        t_tab = _stage_a(emb, msg_W1[l, :_EMB], msg_W1[l, _EMB:],
                         msg_b1[l].reshape(1, -1))
        m1x = _gather_msg(t_tab, dsil)
        p0, p1 = _stage_c(m1x, sg1, sg2, msg_W2[l],
                          msg_b2[l].reshape(1, -1))
        aggr = _scatter_aggr(p0, p1, rows4)
        unm = _unmix(aggr.reshape(2 * _AROWS, 128), hm)
        a0 = unm[:_AROWS].reshape(_NPAD, 32)
        a1 = unm[_AROWS:].reshape(_NPAD, 32)
        emb = _stage_e(emb, a0, a1, upd_W1[l, :_EMB], upd_W1[l, _EMB:],
                       upd_b1[l].reshape(1, -1), upd_W2[l],
                       upd_b2[l].reshape(1, -1))
    out = _pool(emb, pred_W, pred_b.reshape(1, 1))
    return out.reshape(-1)


# speedup vs baseline: 2.0369x; 1.0274x over previous
import functools

import jax
import jax.numpy as jnp
from jax import lax
from jax.experimental import pallas as pl
from jax.experimental.pallas import tpu as pltpu
from jax.experimental.pallas import tpu_sc as plsc

_N = 50000
_E = 800000
_EMB = 64
_L = 4

_NPAD = 50176
_EPAD = 802816
_ROWB = 1024
_EROWB = 2048

_mesh = plsc.VectorSubcoreMesh(core_axis_name="c", subcore_axis_name="s")


def _dot16(a, b):
    return jnp.dot(a.astype(jnp.bfloat16), b.astype(jnp.bfloat16),
                   preferred_element_type=jnp.float32)

_GBLK = 128
_GNBLK = _EPAD // (32 * _GBLK)


@functools.partial(
    pl.kernel,
    out_type=jax.ShapeDtypeStruct((_EPAD, 128), jnp.float32),
    mesh=_mesh,
    scratch_types=[
        pltpu.VMEM((1, 2, 128), jnp.int32),
        pltpu.VMEM((1, 2, 128), jnp.int32),
        pltpu.VMEM((_GBLK, 128), jnp.float32),
        pltpu.VMEM((_GBLK, 128), jnp.float32),
        pltpu.VMEM((_GBLK, 128), jnp.float32),
        pltpu.VMEM((_GBLK, 128), jnp.float32),
        pltpu.VMEM((_GBLK, 128), jnp.float32),
        pltpu.VMEM((_GBLK, 128), jnp.float32),
        pltpu.SemaphoreType.DMA,
        pltpu.SemaphoreType.DMA,
        pltpu.SemaphoreType.DMA,
        pltpu.SemaphoreType.DMA,
        pltpu.SemaphoreType.DMA,
        pltpu.SemaphoreType.DMA,
    ],
)
def _gather_msg(t_hbm, dsil_hbm, out_hbm,
                idxd0, idxd1,
                bufd0, bufs0, bufd1, bufs1, bufo0, bufo1,
                sa0, sb0, sa1, sb1, sw0, sw1):
    c = lax.axis_index("c")
    s = lax.axis_index("s")
    wid = s * 2 + c
    base = wid * _GNBLK
    sets = ((idxd0, bufd0, bufs0, bufo0, sa0, sb0, sw0),
            (idxd1, bufd1, bufs1, bufo1, sa1, sb1, sw1))

    def prep(step, st):
        idxd, bufd, bufs, _, sa, sb, _2 = st
        ch = base + step
        pltpu.sync_copy(dsil_hbm.at[pl.ds(ch, 1)], idxd)
        pltpu.async_copy(t_hbm.at[idxd.at[0, 0]], bufd, sa)
        pltpu.async_copy(t_hbm.at[idxd.at[0, 1]], bufs, sb)

    prep(0, sets[0])
    prep(1, sets[1])

    def outer(i2, carry):
        for p in range(2):
            st = sets[p]
            idxd, bufd, bufs, bufo, sa, sb, sw = st
            i = 2 * i2 + p
            ch = base + i

            @pl.when(i >= 2)
            def _():
                pltpu.make_async_copy(bufo, out_hbm.at[pl.ds(0, _GBLK)],
                                      sw).wait()

            pltpu.make_async_copy(t_hbm.at[idxd.at[0, 0]], bufd, sa).wait()
            pltpu.make_async_copy(t_hbm.at[idxd.at[0, 1]], bufs, sb).wait()

            @plsc.parallel_loop(0, _GBLK, unroll=8)
            def _(r):
                for g in range(0, _EMB, 16):
                    va = bufd[r, pl.ds(g, 16)]
                    vb = bufs[r, pl.ds(_EMB + g, 16)]
                    bufo[r, pl.ds(g, 16)] = jnp.maximum(va + vb, 0.0)
            pltpu.async_copy(bufo, out_hbm.at[pl.ds(ch * _GBLK, _GBLK)], sw)

            @pl.when(i + 2 < _GNBLK)
            def _():
                prep(i + 2, st)
        return carry

    lax.fori_loop(0, _GNBLK // 2, outer, 0)
    pltpu.make_async_copy(bufo0, out_hbm.at[pl.ds(0, _GBLK)], sw0).wait()
    pltpu.make_async_copy(bufo1, out_hbm.at[pl.ds(0, _GBLK)], sw1).wait()


_SBLK = 128
_SNBLK = _EPAD // (16 * _SBLK)
_AROWS = _NPAD // 4
_TAROWS = _AROWS // 16


@functools.partial(
    pl.kernel,
    out_type=jax.ShapeDtypeStruct((2, _AROWS, 128), jnp.float32),
    mesh=_mesh,
    scratch_types=[
        pltpu.VMEM((1, 4, 128), jnp.int32),
        pltpu.VMEM((_SBLK, 128), jnp.float32),
        pltpu.VMEM_SHARED((_AROWS, 128), jnp.float32),
        pltpu.SemaphoreType.DMA,
    ],
)
def _scatter_aggr(p0_hbm, p1_hbm, rows_hbm, out_hbm, idxd, bufp0,
                  acc_sh, sl0):
    c = lax.axis_index("c")
    s = lax.axis_index("s")

    def zrow(r, _):
        for g in range(0, 128, 16):
            bufp0[r, pl.ds(g, 16)] = jnp.zeros((16,), jnp.float32)
        return 0

    lax.fori_loop(0, _SBLK, zrow, 0)
    row0 = s * _TAROWS

    def zcp(k, _):
        pltpu.sync_copy(bufp0.at[pl.ds(0, 16)],
                        acc_sh.at[pl.ds(pl.multiple_of(row0 + k * 16, 8),
                                        16)])
        return 0

    lax.fori_loop(0, 49, zcp, 0)
    plsc.subcore_barrier()

    def one_core(p_hbm):
        base = s * _SNBLK

        def block(bi, carry):
            ch = base + bi
            e0 = ch * 128

            @pl.when(bi % 4 == 0)
            def _():
                pltpu.sync_copy(rows_hbm.at[pl.ds(ch // 4, 1)], idxd)

            pltpu.sync_copy(p_hbm.at[pl.ds(e0, _SBLK)], bufp0)
            pltpu.sync_copy(bufp0, acc_sh.at[idxd.at[0, bi % 4]], add=True)
            return carry

        lax.fori_loop(0, _SNBLK, block, 0)

    @pl.when(c == 0)
    def _():
        one_core(p0_hbm)

    @pl.when(c == 1)
    def _():
        one_core(p1_hbm)

    plsc.subcore_barrier()

    def ocp(k, _):
        r = pl.multiple_of(row0 + k * 16, 8)
        pltpu.sync_copy(acc_sh.at[pl.ds(r, 16)],
                        out_hbm.at[c].at[pl.ds(r, 16)])
        return 0

    lax.fori_loop(0, 49, ocp, 0)


def _lin_in_body(x_ref, w_ref, b_ref, o_ref):
    o_ref[...] = _dot16(x_ref[...], w_ref[...]) + b_ref[...]


def _stage_a_body(h_ref, wa_ref, wb_ref, b_ref, t_ref):
    h = h_ref[...]
    a = _dot16(h, wa_ref[...]) + b_ref[...]
    b = _dot16(h, wb_ref[...])
    t_ref[...] = jnp.concatenate([a, b], axis=1)


def _stage_c_body(m1_ref, s1_ref, s2_ref, w_ref, b_ref, o0_ref, o1_ref):
    m1 = m1_ref[:, :_EMB]
    m = jnp.maximum(_dot16(m1, w_ref[...]) + b_ref[...], 0.0)
    s1 = s1_ref[...]
    s2 = s2_ref[...]
    s3 = s1 * s2
    for c, o_ref in ((0, o0_ref), (1, o1_ref)):
        half = m[:, 32 * c:32 * c + 32]
        o_ref[...] = jnp.concatenate(
            [half, half * s1, half * s2, half * s3], axis=1)


def _unmix_body(acc_ref, hm_ref, o_ref):
    o_ref[...] = jnp.dot(acc_ref[...], hm_ref[...],
                         preferred_element_type=jnp.float32,
                         precision=lax.Precision.HIGHEST)


def _stage_e_body(h_ref, a0_ref, a1_ref, ua_ref, ub_ref, c1_ref, u2_ref,
                  c2_ref, o_ref):
    h = h_ref[...]
    ag = jnp.concatenate([a0_ref[...], a1_ref[...]], axis=1)
    u1 = jnp.maximum(_dot16(h, ua_ref[...]) + _dot16(ag, ub_ref[...])
                     + c1_ref[...], 0.0)
    u = jnp.maximum(_dot16(u1, u2_ref[...]) + c2_ref[...], 0.0)
    o_ref[...] = h + u


def _pool_body(h_ref, pw_ref, pb_ref, o_ref, acc_ref):
    i = pl.program_id(0)

    @pl.when(i == 0)
    def _():
        acc_ref[...] = jnp.zeros_like(acc_ref)

    rows = jax.lax.broadcasted_iota(jnp.int32, (_ROWB, 1), 0) + i * _ROWB
    mask = (rows < _N).astype(jnp.float32)
    acc_ref[...] += jnp.sum(h_ref[...] * mask, axis=0, keepdims=True)

    @pl.when(i == pl.num_programs(0) - 1)
    def _():
        o_ref[...] = (_dot16(acc_ref[...] / float(_N), pw_ref[...])
                      + pb_ref[...])


def _row_spec(blk, ncol):
    return pl.BlockSpec((blk, ncol), lambda i: (i, 0))


def _full_spec(shape):
    return pl.BlockSpec(shape, lambda i: tuple(0 for _ in shape))


def _lin_in(xp, w, b):
    return pl.pallas_call(
        _lin_in_body,
        grid=(_NPAD // _ROWB,),
        in_specs=[_row_spec(_ROWB, 8), _full_spec((8, _EMB)),
                  _full_spec((1, _EMB))],
        out_specs=_row_spec(_ROWB, _EMB),
        out_shape=jax.ShapeDtypeStruct((_NPAD, _EMB), jnp.float32),
    )(xp, w, b)


def _stage_a(h, wa, wb, b):
    return pl.pallas_call(
        _stage_a_body,
        grid=(_NPAD // _ROWB,),
        in_specs=[_row_spec(_ROWB, _EMB), _full_spec((_EMB, _EMB)),
                  _full_spec((_EMB, _EMB)), _full_spec((1, _EMB))],
        out_specs=_row_spec(_ROWB, 2 * _EMB),
        out_shape=jax.ShapeDtypeStruct((_NPAD, 2 * _EMB), jnp.float32),
    )(h, wa, wb, b)


def _stage_c(m1x, sg1, sg2, w, b):
    return pl.pallas_call(
        _stage_c_body,
        grid=(_EPAD // _EROWB,),
        in_specs=[_row_spec(_EROWB, 128), _row_spec(_EROWB, 1),
                  _row_spec(_EROWB, 1),
                  _full_spec((_EMB, _EMB)), _full_spec((1, _EMB))],
        out_specs=[_row_spec(_EROWB, 128), _row_spec(_EROWB, 128)],
        out_shape=[jax.ShapeDtypeStruct((_EPAD, 128), jnp.float32),
                   jax.ShapeDtypeStruct((_EPAD, 128), jnp.float32)],
    )(m1x, sg1, sg2, w, b)


def _unmix(acc2, hm):
    return pl.pallas_call(
        _unmix_body,
        grid=(2 * _AROWS // 896,),
        in_specs=[_row_spec(896, 128), _full_spec((128, 128))],
        out_specs=_row_spec(896, 128),
        out_shape=jax.ShapeDtypeStruct((2 * _AROWS, 128), jnp.float32),
    )(acc2, hm)


def _stage_e(h, a0, a1, ua, ub, c1, u2, c2):
    return pl.pallas_call(
        _stage_e_body,
        grid=(_NPAD // _ROWB,),
        in_specs=[_row_spec(_ROWB, _EMB), _row_spec(_ROWB, 32),
                  _row_spec(_ROWB, 32),
                  _full_spec((_EMB, _EMB)), _full_spec((_EMB, _EMB)),
                  _full_spec((1, _EMB)), _full_spec((_EMB, _EMB)),
                  _full_spec((1, _EMB))],
        out_specs=_row_spec(_ROWB, _EMB),
        out_shape=jax.ShapeDtypeStruct((_NPAD, _EMB), jnp.float32),
    )(h, a0, a1, ua, ub, c1, u2, c2)


def _pool(h, pw, pb):
    return pl.pallas_call(
        _pool_body,
        grid=(_NPAD // _ROWB,),
        in_specs=[_row_spec(_ROWB, _EMB), _full_spec((_EMB, 1)),
                  _full_spec((1, 1))],
        out_specs=_full_spec((1, 1)),
        out_shape=jax.ShapeDtypeStruct((1, 1), jnp.float32),
        scratch_shapes=[pltpu.VMEM((1, _EMB), jnp.float32)],
    )(h, pw, pb)


def kernel(x, edge_index, batch, lin_in_W, lin_in_b,
           msg_W1, msg_b1, msg_W2, msg_b2,
           upd_W1, upd_b1, upd_W2, upd_b2,
           pred_W, pred_b):
    src = edge_index[0]
    dst = edge_index[1]
    pad_e = _EPAD - _E
    dstp = jnp.concatenate(
        [dst, jnp.full((pad_e,), _N, jnp.int32)]).reshape(_EPAD // 128, 128)
    srcp = jnp.concatenate(
        [src, jnp.full((pad_e,), _N, jnp.int32)]).reshape(_EPAD // 128, 128)
    dsil = jnp.stack([dstp, srcp], axis=1)
    rows4 = (dstp >> 2).reshape(_EPAD // 512, 4, 128)
    sg1 = (1 - 2 * (dstp & 1)).astype(jnp.float32).reshape(_EPAD, 1)
    sg2 = (1 - (dstp & 2)).astype(jnp.float32).reshape(_EPAD, 1)
    import numpy as _np
    h4 = _np.array([[1, 1, 1, 1], [1, -1, 1, -1],
                    [1, 1, -1, -1], [1, -1, -1, 1]], dtype=_np.float32)
    hm = jnp.asarray(_np.kron(h4 / 4.0, _np.eye(32, dtype=_np.float32)))

    xp = jnp.pad(x, ((0, _NPAD - _N), (0, 1)))
    w_in = jnp.pad(lin_in_W, ((0, 1), (0, 0)))

    emb = _lin_in(xp, w_in, lin_in_b.reshape(1, -1))
    for l in range(_L):
        t_tab = _stage_a(emb, msg_W1[l, :_EMB], msg_W1[l, _EMB:],
                         msg_b1[l].reshape(1, -1))
        m1x = _gather_msg(t_tab, dsil)
        p0, p1 = _stage_c(m1x, sg1, sg2, msg_W2[l],
                          msg_b2[l].reshape(1, -1))
        aggr = _scatter_aggr(p0, p1, rows4)
        unm = _unmix(aggr.reshape(2 * _AROWS, 128), hm)
        a0 = unm[:_AROWS].reshape(_NPAD, 32)
        a1 = unm[_AROWS:].reshape(_NPAD, 32)
        emb = _stage_e(emb, a0, a1, upd_W1[l, :_EMB], upd_W1[l, _EMB:],
                       upd_b1[l].reshape(1, -1), upd_W2[l],
                       upd_b2[l].reshape(1, -1))
    out = _pool(emb, pred_W, pred_b.reshape(1, 1))
    return out.reshape(-1)
